# Initial kernel scaffold; baseline (speedup 1.0000x reference)
import jax
import jax.numpy as jnp
from jax.experimental import pallas as pl


def kernel(x, x0, edge_index, norm, W):
    raise NotImplementedError("write your pallas kernel here")


# speedup vs baseline: 3.4389x; 3.4389x over previous
import math

import jax
import jax.numpy as jnp
from jax import lax
from jax.experimental import pallas as pl
from jax.experimental.pallas import tpu as pltpu
from jax.experimental.pallas import tpu_sc as plsc

_ALPHA = 0.1
_BETA = math.log(0.5 / 1 + 1.0)

_N_NODES = 10000
_N_PAD = 10240
_HIDDEN = 128

_NC = 2
_NS = 16
_NW = _NC * _NS
_CHUNK = 128
_CHUNKS_PER_W = 80
_EDGES_PER_W = _CHUNK * _CHUNKS_PER_W
_EP = _NW * _EDGES_PER_W
_ROWS_PER_TILE = _N_PAD // _NS
_ZCOPY = 128


def _sc_aggregate(x, row3, col3, norm3):
    mesh = plsc.VectorSubcoreMesh(core_axis_name="c", subcore_axis_name="s")

    def body(x_hbm, row_hbm, col_hbm, norm_hbm, out_hbm,
             row_v, col_v, norm_v, rows_v, acc, sem):
        c = lax.axis_index("c")
        s = lax.axis_index("s")
        wid = s * _NC + c

        def zrow(e, carry):
            for f in range(_HIDDEN // 16):
                rows_v[e, pl.ds(f * 16, 16)] = jnp.zeros((16,), jnp.float32)
            return carry
        lax.fori_loop(0, _CHUNK, zrow, 0)
        base = s * _ROWS_PER_TILE
        for i in range(_ROWS_PER_TILE // _ZCOPY):
            pltpu.sync_copy(rows_v, acc.at[pl.ds(base + i * _ZCOPY, _ZCOPY)])
        plsc.subcore_barrier()

        pltpu.sync_copy(row_hbm.at[wid], row_v)
        pltpu.sync_copy(col_hbm.at[wid], col_v)
        pltpu.sync_copy(norm_hbm.at[wid], norm_v)

        def chunk(k, carry):
            pltpu.async_copy(x_hbm.at[row_v.at[k]], rows_v, sem).wait()

            def scale(g, c2):
                n16 = norm_v[k, pl.ds(g * 16, 16)]
                for j in range(16):
                    e = g * 16 + j
                    b = jnp.full((16,), n16[j], jnp.float32)
                    for f in range(_HIDDEN // 16):
                        sl = pl.ds(f * 16, 16)
                        rows_v[e, sl] = rows_v[e, sl] * b
                return c2
            lax.fori_loop(0, _CHUNK // 16, scale, 0)

            pltpu.sync_copy(rows_v, acc.at[col_v.at[k]], add=True)
            return carry
        lax.fori_loop(0, _CHUNKS_PER_W, chunk, 0)

        plsc.subcore_barrier()
        pltpu.sync_copy(acc.at[pl.ds(base, _ROWS_PER_TILE)],
                        out_hbm.at[c, pl.ds(base, _ROWS_PER_TILE)])

    return pl.kernel(
        body,
        out_type=jax.ShapeDtypeStruct((_NC, _N_PAD, _HIDDEN), jnp.float32),
        mesh=mesh,
        scratch_types=[
            pltpu.VMEM((_CHUNKS_PER_W, _CHUNK), jnp.int32),
            pltpu.VMEM((_CHUNKS_PER_W, _CHUNK), jnp.int32),
            pltpu.VMEM((_CHUNKS_PER_W, _CHUNK), jnp.float32),
            pltpu.VMEM((_CHUNK, _HIDDEN), jnp.float32),
            pltpu.VMEM_SHARED((_N_PAD, _HIDDEN), jnp.float32),
            pltpu.SemaphoreType.DMA,
        ],
    )(x, row3, col3, norm3)


def _combine_body(p_ref, x0_ref, w_ref, o_ref):
    p = p_ref[...]
    h = (1.0 - _ALPHA) * (p[0] + p[1]) + _ALPHA * x0_ref[...]
    hw = lax.dot_general(h, w_ref[...], (((1,), (1,)), ((), ())),
                         preferred_element_type=jnp.float32)
    o_ref[...] = (1.0 - _BETA) * h + _BETA * hw


def _tc_combine(partials, x0p, W):
    blk = 640
    grid = _N_PAD // blk
    return pl.pallas_call(
        _combine_body,
        grid=(grid,),
        in_specs=[
            pl.BlockSpec((_NC, blk, _HIDDEN), lambda i: (0, i, 0)),
            pl.BlockSpec((blk, _HIDDEN), lambda i: (i, 0)),
            pl.BlockSpec((_HIDDEN, _HIDDEN), lambda i: (0, 0)),
        ],
        out_specs=pl.BlockSpec((blk, _HIDDEN), lambda i: (i, 0)),
        out_shape=jax.ShapeDtypeStruct((_N_PAD, _HIDDEN), jnp.float32),
    )(partials, x0p, W)


def kernel(x, x0, edge_index, norm, W):
    row = edge_index[0].astype(jnp.int32)
    col = edge_index[1].astype(jnp.int32)
    e = row.shape[0]
    pad = _EP - e
    row3 = jnp.concatenate([row, jnp.zeros((pad,), jnp.int32)]).reshape(
        _NW, _CHUNKS_PER_W, _CHUNK)
    col3 = jnp.concatenate([col, jnp.zeros((pad,), jnp.int32)]).reshape(
        _NW, _CHUNKS_PER_W, _CHUNK)
    norm3 = jnp.concatenate(
        [norm.astype(jnp.float32), jnp.zeros((pad,), jnp.float32)]).reshape(
        _NW, _CHUNKS_PER_W, _CHUNK)
    partials = _sc_aggregate(x, row3, col3, norm3)
    x0p = jnp.concatenate(
        [x0, jnp.zeros((_N_PAD - _N_NODES, _HIDDEN), jnp.float32)])
    return _tc_combine(partials, x0p, W)[: _N_NODES]


# speedup vs baseline: 4.1208x; 1.1983x over previous
import math

import jax
import jax.numpy as jnp
from jax import lax
from jax.experimental import pallas as pl
from jax.experimental.pallas import tpu as pltpu
from jax.experimental.pallas import tpu_sc as plsc

_ALPHA = 0.1
_BETA = math.log(0.5 / 1 + 1.0)

_N_NODES = 10000
_N_PAD = 10240
_HIDDEN = 128

_NC = 2
_NS = 16
_NW = _NC * _NS
_CHUNK = 128
_CHUNKS_PER_W = 80
_EDGES_PER_W = _CHUNK * _CHUNKS_PER_W
_EP = _NW * _EDGES_PER_W
_ROWS_PER_TILE = _N_PAD // _NS
_ZCOPY = 128


def _sc_aggregate(x, ed):
    mesh = plsc.VectorSubcoreMesh(core_axis_name="c", subcore_axis_name="s")

    def body(x_hbm, ed_hbm, out_hbm, rows2, ering, acc,
             g0, g1, e0, e1, e2, e3):
        c = lax.axis_index("c")
        s = lax.axis_index("s")
        wid = s * _NC + c
        gsems = (g0, g1)
        esems = (e0, e1, e2, e3)

        def ecopy(k, slot):
            pltpu.async_copy(ed_hbm.at[wid, k], ering.at[slot], esems[slot])

        def ewait(k, slot):
            pltpu.make_async_copy(ed_hbm.at[wid, k], ering.at[slot],
                                  esems[slot]).wait()

        def gather(slot, b):
            pltpu.async_copy(x_hbm.at[ering.at[slot, 0]], rows2.at[b],
                             gsems[b])

        def gwait(slot, b):
            pltpu.make_async_copy(x_hbm.at[ering.at[slot, 0]], rows2.at[b],
                                  gsems[b]).wait()

        def scatter(slot, b):
            pltpu.sync_copy(rows2.at[b], acc.at[ering.at[slot, 1]],
                            add=True)

        def scale(slot, b):
            def sg(g, c2):
                n16 = lax.bitcast_convert_type(
                    ering[slot, 2, pl.ds(g * 16, 16)], jnp.float32)
                for jj in range(16):
                    e = g * 16 + jj
                    nb = jnp.full((16,), n16[jj], jnp.float32)
                    for f in range(_HIDDEN // 16):
                        sl = pl.ds(f * 16, 16)
                        rows2[b, e, sl] = rows2[b, e, sl] * nb
                return c2
            lax.fori_loop(0, _CHUNK // 16, sg, 0)

        def zrow(e, carry):
            for f in range(_HIDDEN // 16):
                rows2[0, e, pl.ds(f * 16, 16)] = jnp.zeros((16,), jnp.float32)
            return carry
        lax.fori_loop(0, _CHUNK, zrow, 0)
        base = s * _ROWS_PER_TILE
        for i in range(_ROWS_PER_TILE // _ZCOPY):
            pltpu.sync_copy(rows2.at[0],
                            acc.at[pl.ds(base + i * _ZCOPY, _ZCOPY)])
        plsc.subcore_barrier()

        ecopy(0, 0)
        ecopy(1, 1)
        ecopy(2, 2)
        ewait(0, 0)
        gather(0, 0)

        def quad(m, carry):
            for j in range(4):
                slot = j
                b = j % 2
                nslot = (j + 1) % 4
                nb_ = (j + 1) % 2
                gwait(slot, b)
                if j < 3:
                    ewait(4 * m + j + 1, nslot)
                    gather(nslot, nb_)
                else:
                    @pl.when(m < _CHUNKS_PER_W // 4 - 1)
                    def _():
                        ewait(4 * m + 4, nslot)
                        gather(nslot, nb_)
                if j == 0:
                    ecopy(4 * m + 3, 3)
                else:
                    @pl.when(m < _CHUNKS_PER_W // 4 - 1)
                    def _():
                        ecopy(4 * m + j + 3, (j + 3) % 4)
                scale(slot, b)
                scatter(slot, b)
            return carry
        lax.fori_loop(0, _CHUNKS_PER_W // 4, quad, 0)

        plsc.subcore_barrier()
        pltpu.sync_copy(acc.at[pl.ds(base, _ROWS_PER_TILE)],
                        out_hbm.at[c, pl.ds(base, _ROWS_PER_TILE)])

    return pl.kernel(
        body,
        out_type=jax.ShapeDtypeStruct((_NC, _N_PAD, _HIDDEN), jnp.float32),
        mesh=mesh,
        scratch_types=[
            pltpu.VMEM((2, _CHUNK, _HIDDEN), jnp.float32),
            pltpu.VMEM((4, 3, _CHUNK), jnp.int32),
            pltpu.VMEM_SHARED((_N_PAD, _HIDDEN), jnp.float32),
            pltpu.SemaphoreType.DMA,
            pltpu.SemaphoreType.DMA,
            pltpu.SemaphoreType.DMA,
            pltpu.SemaphoreType.DMA,
            pltpu.SemaphoreType.DMA,
            pltpu.SemaphoreType.DMA,
        ],
    )(x, ed)


def _combine_body(p_ref, x0_ref, w_ref, o_ref):
    p = p_ref[...]
    h = (1.0 - _ALPHA) * (p[0] + p[1]) + _ALPHA * x0_ref[...]
    hw = lax.dot_general(h, w_ref[...], (((1,), (1,)), ((), ())),
                         preferred_element_type=jnp.float32)
    o_ref[...] = (1.0 - _BETA) * h + _BETA * hw


def _tc_combine(partials, x0p, W):
    blk = 640
    grid = _N_PAD // blk
    return pl.pallas_call(
        _combine_body,
        grid=(grid,),
        in_specs=[
            pl.BlockSpec((_NC, blk, _HIDDEN), lambda i: (0, i, 0)),
            pl.BlockSpec((blk, _HIDDEN), lambda i: (i, 0)),
            pl.BlockSpec((_HIDDEN, _HIDDEN), lambda i: (0, 0)),
        ],
        out_specs=pl.BlockSpec((blk, _HIDDEN), lambda i: (i, 0)),
        out_shape=jax.ShapeDtypeStruct((_N_PAD, _HIDDEN), jnp.float32),
    )(partials, x0p, W)


def kernel(x, x0, edge_index, norm, W):
    row = edge_index[0].astype(jnp.int32)
    col = edge_index[1].astype(jnp.int32)
    e = row.shape[0]
    pad = _EP - e
    row_p = jnp.concatenate([row, jnp.zeros((pad,), jnp.int32)])
    col_p = jnp.concatenate([col, jnp.zeros((pad,), jnp.int32)])
    norm_p = jnp.concatenate(
        [lax.bitcast_convert_type(norm.astype(jnp.float32), jnp.int32),
         jnp.zeros((pad,), jnp.int32)])
    ed = jnp.stack([row_p, col_p, norm_p], axis=0).reshape(
        3, _NW, _CHUNKS_PER_W, _CHUNK).transpose(1, 2, 0, 3)
    partials = _sc_aggregate(x, ed)
    x0p = jnp.concatenate(
        [x0, jnp.zeros((_N_PAD - _N_NODES, _HIDDEN), jnp.float32)])
    return _tc_combine(partials, x0p, W)[: _N_NODES]


# speedup vs baseline: 4.1975x; 1.0186x over previous
import math

import jax
import jax.numpy as jnp
from jax import lax
from jax.experimental import pallas as pl
from jax.experimental.pallas import tpu as pltpu
from jax.experimental.pallas import tpu_sc as plsc

_ALPHA = 0.1
_BETA = math.log(0.5 / 1 + 1.0)

_N_NODES = 10000
_N_PAD = 10240
_HIDDEN = 128

_NC = 2
_NS = 16
_NW = _NC * _NS
_CHUNK = 80
_CHUNKS_PER_W = 128
_EDGES_PER_W = _CHUNK * _CHUNKS_PER_W
_EP = _NW * _EDGES_PER_W
_ROWS_PER_TILE = _N_PAD // _NS
_ZCOPY = 80


def _sc_aggregate(x, ed):
    mesh = plsc.VectorSubcoreMesh(core_axis_name="c", subcore_axis_name="s")

    def body(x_hbm, ed_hbm, out_hbm, rows4, ering, acc,
             g0, g1, g2, g3, e0, e1, e2, e3, e4, e5, e6, e7):
        c = lax.axis_index("c")
        s = lax.axis_index("s")
        wid = s * _NC + c
        gsems = (g0, g1, g2, g3)
        esems = (e0, e1, e2, e3, e4, e5, e6, e7)

        def ecopy(k, slot):
            pltpu.async_copy(ed_hbm.at[wid, k], ering.at[slot], esems[slot])

        def ewait(k, slot):
            pltpu.make_async_copy(ed_hbm.at[wid, k], ering.at[slot],
                                  esems[slot]).wait()

        def gather(slot, b):
            pltpu.async_copy(x_hbm.at[ering.at[slot, 0]], rows4.at[b],
                             gsems[b])

        def gwait(slot, b):
            pltpu.make_async_copy(x_hbm.at[ering.at[slot, 0]], rows4.at[b],
                                  gsems[b]).wait()

        def scatter(slot, b):
            pltpu.sync_copy(rows4.at[b], acc.at[ering.at[slot, 1]],
                            add=True)

        def scale(slot, b):
            def sg(g, c2):
                n16 = lax.bitcast_convert_type(
                    ering[slot, 2, pl.ds(g * 16, 16)], jnp.float32)
                for jj in range(16):
                    e = g * 16 + jj
                    nb = jnp.full((16,), n16[jj], jnp.float32)
                    for f in range(_HIDDEN // 16):
                        sl = pl.ds(f * 16, 16)
                        rows4[b, e, sl] = rows4[b, e, sl] * nb
                return c2
            lax.fori_loop(0, _CHUNK // 16, sg, 0)

        def zrow(e, carry):
            for f in range(_HIDDEN // 16):
                rows4[0, e, pl.ds(f * 16, 16)] = jnp.zeros((16,), jnp.float32)
            return carry
        lax.fori_loop(0, _CHUNK, zrow, 0)
        base = s * _ROWS_PER_TILE
        for i in range(_ROWS_PER_TILE // _ZCOPY):
            pltpu.sync_copy(rows4.at[0],
                            acc.at[pl.ds(base + i * _ZCOPY, _ZCOPY)])
        plsc.subcore_barrier()

        for kk in range(6):
            ecopy(kk, kk)
        for kk in range(3):
            ewait(kk, kk)
            gather(kk, kk)

        _M = _CHUNKS_PER_W // 8
        def oct_(m, carry):
            for j in range(8):
                slot = j
                b = j % 4
                gwait(slot, b)
                gslot = (j + 3) % 8
                gb = (j + 3) % 4
                if j < 5:
                    ewait(8 * m + j + 3, gslot)
                    gather(gslot, gb)
                else:
                    @pl.when(m < _M - 1)
                    def _():
                        ewait(8 * m + j + 3, gslot)
                        gather(gslot, gb)
                if j < 2:
                    ecopy(8 * m + j + 6, (j + 6) % 8)
                else:
                    @pl.when(m < _M - 1)
                    def _():
                        ecopy(8 * m + j + 6, (j + 6) % 8)
                scale(slot, b)
                scatter(slot, b)
            return carry
        lax.fori_loop(0, _M, oct_, 0)

        plsc.subcore_barrier()
        pltpu.sync_copy(acc.at[pl.ds(base, _ROWS_PER_TILE)],
                        out_hbm.at[c, pl.ds(base, _ROWS_PER_TILE)])

    return pl.kernel(
        body,
        out_type=jax.ShapeDtypeStruct((_NC, _N_PAD, _HIDDEN), jnp.float32),
        mesh=mesh,
        scratch_types=[
            pltpu.VMEM((4, _CHUNK, _HIDDEN), jnp.float32),
            pltpu.VMEM((8, 3, _CHUNK), jnp.int32),
            pltpu.VMEM_SHARED((_N_PAD, _HIDDEN), jnp.float32),
        ] + [pltpu.SemaphoreType.DMA] * 12,
    )(x, ed)


def _combine_body(p_ref, x0_ref, w_ref, o_ref):
    p = p_ref[...]
    h = (1.0 - _ALPHA) * (p[0] + p[1]) + _ALPHA * x0_ref[...]
    hw = lax.dot_general(h, w_ref[...], (((1,), (1,)), ((), ())),
                         preferred_element_type=jnp.float32)
    o_ref[...] = (1.0 - _BETA) * h + _BETA * hw


def _tc_combine(partials, x0p, W):
    blk = 640
    grid = _N_PAD // blk
    return pl.pallas_call(
        _combine_body,
        grid=(grid,),
        in_specs=[
            pl.BlockSpec((_NC, blk, _HIDDEN), lambda i: (0, i, 0)),
            pl.BlockSpec((blk, _HIDDEN), lambda i: (i, 0)),
            pl.BlockSpec((_HIDDEN, _HIDDEN), lambda i: (0, 0)),
        ],
        out_specs=pl.BlockSpec((blk, _HIDDEN), lambda i: (i, 0)),
        out_shape=jax.ShapeDtypeStruct((_N_PAD, _HIDDEN), jnp.float32),
    )(partials, x0p, W)


def kernel(x, x0, edge_index, norm, W):
    row = edge_index[0].astype(jnp.int32)
    col = edge_index[1].astype(jnp.int32)
    e = row.shape[0]
    pad = _EP - e
    row_p = jnp.concatenate([row, jnp.zeros((pad,), jnp.int32)])
    col_p = jnp.concatenate([col, jnp.zeros((pad,), jnp.int32)])
    norm_p = jnp.concatenate(
        [lax.bitcast_convert_type(norm.astype(jnp.float32), jnp.int32),
         jnp.zeros((pad,), jnp.int32)])
    ed = jnp.stack([row_p, col_p, norm_p], axis=0).reshape(
        3, _NW, _CHUNKS_PER_W, _CHUNK).transpose(1, 2, 0, 3)
    partials = _sc_aggregate(x, ed)
    x0p = jnp.concatenate(
        [x0, jnp.zeros((_N_PAD - _N_NODES, _HIDDEN), jnp.float32)])
    return _tc_combine(partials, x0p, W)[: _N_NODES]


# speedup vs baseline: 4.6639x; 1.1111x over previous
import math

import jax
import jax.numpy as jnp
from jax import lax
from jax.experimental import pallas as pl
from jax.experimental.pallas import tpu as pltpu
from jax.experimental.pallas import tpu_sc as plsc

_ALPHA = 0.1
_BETA = math.log(0.5 / 1 + 1.0)

_N_NODES = 10000
_N_PAD = 10240
_HIDDEN = 128

_NC = 2
_NS = 16
_NW = _NC * _NS
_CHUNK = 80
_CHUNKS_PER_W = 128
_EDGES_PER_W = _CHUNK * _CHUNKS_PER_W
_EP = _NW * _EDGES_PER_W
_ROWS_PER_TILE = _N_PAD // _NS
_ZCOPY = 80


def _sc_aggregate(x, ed):
    mesh = plsc.VectorSubcoreMesh(core_axis_name="c", subcore_axis_name="s")

    def body(x_hbm, ed_hbm, out_hbm, rows4, stag, ering, acc,
             g0, g1, g2, g3, e0, e1, e2, e3, e4, e5, e6, e7):
        c = lax.axis_index("c")
        s = lax.axis_index("s")
        wid = s * _NC + c
        gsems = (g0, g1, g2, g3)
        esems = (e0, e1, e2, e3, e4, e5, e6, e7)

        def ecopy(k, slot):
            pltpu.async_copy(ed_hbm.at[wid, k], ering.at[slot], esems[slot])

        def ewait(k, slot):
            pltpu.make_async_copy(ed_hbm.at[wid, k], ering.at[slot],
                                  esems[slot]).wait()

        def gather(slot, b):
            pltpu.async_copy(x_hbm.at[ering.at[slot, 0]], rows4.at[b],
                             gsems[b])

        def gwait(slot, b):
            pltpu.make_async_copy(x_hbm.at[ering.at[slot, 0]], rows4.at[b],
                                  gsems[b]).wait()

        def scatter(slot, b):
            pltpu.sync_copy(stag, acc.at[ering.at[slot, 1]],
                            add=True)

        def scale(slot, b):
            def sg(g, c2):
                n16 = lax.bitcast_convert_type(
                    ering[slot, 2, pl.ds(g * 16, 16)], jnp.float32)
                for jj in range(16):
                    e = g * 16 + jj
                    nb = jnp.full((16,), n16[jj], jnp.float32)
                    for f in range(_HIDDEN // 32):
                        w = rows4[b, e, pl.ds(f * 16, 16)]
                        lo = lax.bitcast_convert_type(
                            w << 16, jnp.float32)
                        hi = lax.bitcast_convert_type(
                            w & jnp.full((16,), -65536, jnp.int32),
                            jnp.float32)
                        stag[e, pl.ds(f * 32, 16)] = lo * nb
                        stag[e, pl.ds(f * 32 + 16, 16)] = hi * nb
                return c2
            lax.fori_loop(0, _CHUNK // 16, sg, 0)

        def zrow(e, carry):
            for f in range(_HIDDEN // 16):
                stag[e, pl.ds(f * 16, 16)] = jnp.zeros((16,), jnp.float32)
            return carry
        lax.fori_loop(0, _CHUNK, zrow, 0)
        base = s * _ROWS_PER_TILE
        for i in range(_ROWS_PER_TILE // _ZCOPY):
            pltpu.sync_copy(stag,
                            acc.at[pl.ds(base + i * _ZCOPY, _ZCOPY)])
        plsc.subcore_barrier()

        for kk in range(6):
            ecopy(kk, kk)
        for kk in range(3):
            ewait(kk, kk)
            gather(kk, kk)

        _M = _CHUNKS_PER_W // 8
        def oct_(m, carry):
            for j in range(8):
                slot = j
                b = j % 4
                gwait(slot, b)
                gslot = (j + 3) % 8
                gb = (j + 3) % 4
                if j < 5:
                    ewait(8 * m + j + 3, gslot)
                    gather(gslot, gb)
                else:
                    @pl.when(m < _M - 1)
                    def _():
                        ewait(8 * m + j + 3, gslot)
                        gather(gslot, gb)
                if j < 2:
                    ecopy(8 * m + j + 6, (j + 6) % 8)
                else:
                    @pl.when(m < _M - 1)
                    def _():
                        ecopy(8 * m + j + 6, (j + 6) % 8)
                scale(slot, b)
                scatter(slot, b)
            return carry
        lax.fori_loop(0, _M, oct_, 0)

        plsc.subcore_barrier()
        pltpu.sync_copy(acc.at[pl.ds(base, _ROWS_PER_TILE)],
                        out_hbm.at[c, pl.ds(base, _ROWS_PER_TILE)])

    return pl.kernel(
        body,
        out_type=jax.ShapeDtypeStruct((_NC, _N_PAD, _HIDDEN), jnp.float32),
        mesh=mesh,
        compiler_params=pltpu.CompilerParams(use_tc_tiling_on_sc=False),
        scratch_types=[
            pltpu.VMEM((4, _CHUNK, _HIDDEN // 2), jnp.int32),
            pltpu.VMEM((_CHUNK, _HIDDEN), jnp.float32),
            pltpu.VMEM((8, 3, _CHUNK), jnp.int32),
            pltpu.VMEM_SHARED((_N_PAD, _HIDDEN), jnp.float32),
        ] + [pltpu.SemaphoreType.DMA] * 12,
    )(x, ed)


def _combine_body(p_ref, x0_ref, w_ref, o_ref):
    p = p_ref[...]
    h = (1.0 - _ALPHA) * (p[0] + p[1]) + _ALPHA * x0_ref[...]
    hw = lax.dot_general(h, w_ref[...], (((1,), (1,)), ((), ())),
                         preferred_element_type=jnp.float32)
    o_ref[...] = (1.0 - _BETA) * h + _BETA * hw


def _tc_combine(partials, x0p, W):
    blk = 640
    grid = _N_PAD // blk
    return pl.pallas_call(
        _combine_body,
        grid=(grid,),
        in_specs=[
            pl.BlockSpec((_NC, blk, _HIDDEN), lambda i: (0, i, 0)),
            pl.BlockSpec((blk, _HIDDEN), lambda i: (i, 0)),
            pl.BlockSpec((_HIDDEN, _HIDDEN), lambda i: (0, 0)),
        ],
        out_specs=pl.BlockSpec((blk, _HIDDEN), lambda i: (i, 0)),
        out_shape=jax.ShapeDtypeStruct((_N_PAD, _HIDDEN), jnp.float32),
    )(partials, x0p, W)


def kernel(x, x0, edge_index, norm, W):
    row = edge_index[0].astype(jnp.int32)
    col = edge_index[1].astype(jnp.int32)
    e = row.shape[0]
    pad = _EP - e
    row_p = jnp.concatenate([row, jnp.zeros((pad,), jnp.int32)])
    col_p = jnp.concatenate([col, jnp.zeros((pad,), jnp.int32)])
    norm_p = jnp.concatenate(
        [lax.bitcast_convert_type(norm.astype(jnp.float32), jnp.int32),
         jnp.zeros((pad,), jnp.int32)])
    ed = jnp.stack([row_p, col_p, norm_p], axis=0).reshape(
        3, _NW, _CHUNKS_PER_W, _CHUNK).transpose(1, 2, 0, 3)
    xb = lax.bitcast_convert_type(
        x.astype(jnp.bfloat16).reshape(
            _N_NODES, _HIDDEN // 32, 2, 16).transpose(0, 1, 3, 2),
        jnp.int32).reshape(_N_NODES, _HIDDEN // 2)
    partials = _sc_aggregate(xb, ed)
    x0p = jnp.concatenate(
        [x0, jnp.zeros((_N_PAD - _N_NODES, _HIDDEN), jnp.float32)])
    return _tc_combine(partials, x0p, W)[: _N_NODES]


# speedup vs baseline: 5.2497x; 1.1256x over previous
import math

import jax
import jax.numpy as jnp
from jax import lax
from jax.experimental import pallas as pl
from jax.experimental.pallas import tpu as pltpu
from jax.experimental.pallas import tpu_sc as plsc

_ALPHA = 0.1
_BETA = math.log(0.5 / 1 + 1.0)

_N_NODES = 10000
_HIDDEN = 128

_NC = 2
_NS = 16
_NW = _NC * _NS
_CHUNK = 80
_CHUNKS_PER_W = 125
_EDGES_PER_W = _CHUNK * _CHUNKS_PER_W
_NBUF = 5
_ROWS_PER_TILE = 640
_ZCOPY = 80


def _sc_aggregate(xb, ei, nrm):
    mesh = plsc.VectorSubcoreMesh(core_axis_name="c", subcore_axis_name="s")

    def body(x_hbm, ei_hbm, nrm_hbm, out_hbm, rows5, stag, iring, nring, acc,
             g0, g1, g2, g3, g4, e0, e1, e2, e3, e4):
        c = lax.axis_index("c")
        s = lax.axis_index("s")
        wid = s * _NC + c
        gsems = (g0, g1, g2, g3, g4)
        esems = (e0, e1, e2, e3, e4)

        def ecopy(k, slot):
            pltpu.async_copy(ei_hbm.at[0, wid, k], iring.at[slot, 0],
                             esems[slot])
            pltpu.async_copy(ei_hbm.at[1, wid, k], iring.at[slot, 1],
                             esems[slot])
            pltpu.async_copy(nrm_hbm.at[wid, k], nring.at[slot],
                             esems[slot])

        def ewait(k, slot):
            pltpu.make_async_copy(ei_hbm.at[0, wid, k], iring.at[slot, 0],
                                  esems[slot]).wait()
            pltpu.make_async_copy(ei_hbm.at[1, wid, k], iring.at[slot, 1],
                                  esems[slot]).wait()
            pltpu.make_async_copy(nrm_hbm.at[wid, k], nring.at[slot],
                                  esems[slot]).wait()

        def gather(slot):
            pltpu.async_copy(x_hbm.at[iring.at[slot, 0]], rows5.at[slot],
                             gsems[slot])

        def gwait(slot):
            pltpu.make_async_copy(x_hbm.at[iring.at[slot, 0]],
                                  rows5.at[slot], gsems[slot]).wait()

        def scatter(slot):
            pltpu.sync_copy(stag, acc.at[iring.at[slot, 1]], add=True)

        def scale(slot):
            def sg(g, c2):
                n16 = nring[slot, pl.ds(g * 16, 16)]
                for jj in range(16):
                    e = g * 16 + jj
                    nb = jnp.full((16,), n16[jj], jnp.float32)
                    for f in range(_HIDDEN // 32):
                        w = rows5[slot, e, pl.ds(f * 16, 16)]
                        lo = lax.bitcast_convert_type(w << 16, jnp.float32)
                        hi = lax.bitcast_convert_type(
                            w & jnp.full((16,), -65536, jnp.int32),
                            jnp.float32)
                        stag[e, pl.ds(f * 32, 16)] = lo * nb
                        stag[e, pl.ds(f * 32 + 16, 16)] = hi * nb
                return c2
            lax.fori_loop(0, _CHUNK // 16, sg, 0)

        def zrow(e, carry):
            for f in range(_HIDDEN // 16):
                stag[e, pl.ds(f * 16, 16)] = jnp.zeros((16,), jnp.float32)
            return carry
        lax.fori_loop(0, _CHUNK, zrow, 0)
        base = s * _ROWS_PER_TILE

        @pl.when(s < _NS - 1)
        def _():
            for i in range(_ROWS_PER_TILE // _ZCOPY):
                pltpu.sync_copy(stag, acc.at[pl.ds(base + i * _ZCOPY,
                                                   _ZCOPY)])

        @pl.when(s == _NS - 1)
        def _():
            for i in range(5):
                pltpu.sync_copy(
                    stag, acc.at[pl.ds(9600 + i * _ZCOPY, _ZCOPY)])
        plsc.subcore_barrier()

        for kk in range(4):
            ecopy(kk, kk)
        for kk in range(3):
            ewait(kk, kk)
            gather(kk)

        _M = _CHUNKS_PER_W // _NBUF
        def ring(m, carry):
            for j in range(_NBUF):
                gwait(j)
                gslot = (j + 3) % _NBUF
                if j < 2:
                    ewait(5 * m + j + 3, gslot)
                    gather(gslot)
                else:
                    @pl.when(m < _M - 1)
                    def _():
                        ewait(5 * m + j + 3, gslot)
                        gather(gslot)
                if j < 1:
                    ecopy(5 * m + j + 4, (j + 4) % _NBUF)
                else:
                    @pl.when(m < _M - 1)
                    def _():
                        ecopy(5 * m + j + 4, (j + 4) % _NBUF)
                scale(j)
                scatter(j)
            return carry
        lax.fori_loop(0, _M, ring, 0)

        plsc.subcore_barrier()

        @pl.when(s < _NS - 1)
        def _():
            pltpu.sync_copy(acc.at[pl.ds(base, _ROWS_PER_TILE)],
                            out_hbm.at[c, pl.ds(base, _ROWS_PER_TILE)])

        @pl.when(s == _NS - 1)
        def _():
            pltpu.sync_copy(acc.at[pl.ds(9600, 400)],
                            out_hbm.at[c, pl.ds(9600, 400)])

    return pl.kernel(
        body,
        out_type=jax.ShapeDtypeStruct((_NC, _N_NODES, _HIDDEN), jnp.float32),
        mesh=mesh,
        compiler_params=pltpu.CompilerParams(use_tc_tiling_on_sc=False),
        scratch_types=[
            pltpu.VMEM((_NBUF, _CHUNK, _HIDDEN // 2), jnp.int32),
            pltpu.VMEM((_CHUNK, _HIDDEN), jnp.float32),
            pltpu.VMEM((_NBUF, 2, _CHUNK), jnp.int32),
            pltpu.VMEM((_NBUF, _CHUNK), jnp.float32),
            pltpu.VMEM_SHARED((_N_NODES, _HIDDEN), jnp.float32),
        ] + [pltpu.SemaphoreType.DMA] * 10,
    )(xb, ei, nrm)


def _combine_body(p_ref, x0_ref, w_ref, o_ref):
    p = p_ref[...]
    h = (1.0 - _ALPHA) * (p[0] + p[1]) + _ALPHA * x0_ref[...]
    hw = lax.dot_general(h, w_ref[...], (((1,), (1,)), ((), ())),
                         preferred_element_type=jnp.float32)
    o_ref[...] = (1.0 - _BETA) * h + _BETA * hw


def _tc_combine(partials, x0, W):
    blk = 400
    grid = _N_NODES // blk
    return pl.pallas_call(
        _combine_body,
        grid=(grid,),
        in_specs=[
            pl.BlockSpec((_NC, blk, _HIDDEN), lambda i: (0, i, 0)),
            pl.BlockSpec((blk, _HIDDEN), lambda i: (i, 0)),
            pl.BlockSpec((_HIDDEN, _HIDDEN), lambda i: (0, 0)),
        ],
        out_specs=pl.BlockSpec((blk, _HIDDEN), lambda i: (i, 0)),
        out_shape=jax.ShapeDtypeStruct((_N_NODES, _HIDDEN), jnp.float32),
    )(partials, x0, W)


def kernel(x, x0, edge_index, norm, W):
    ei = edge_index.astype(jnp.int32).reshape(
        2, _NW, _CHUNKS_PER_W, _CHUNK)
    nrm = norm.astype(jnp.float32).reshape(_NW, _CHUNKS_PER_W, _CHUNK)
    xb = lax.bitcast_convert_type(
        x.astype(jnp.bfloat16).reshape(
            _N_NODES, _HIDDEN // 32, 2, 16).transpose(0, 1, 3, 2),
        jnp.int32).reshape(_N_NODES, _HIDDEN // 2)
    partials = _sc_aggregate(xb, ei, nrm)
    return _tc_combine(partials, x0, W)


# speedup vs baseline: 5.5343x; 1.0542x over previous
import math

import jax
import jax.numpy as jnp
from jax import lax
from jax.experimental import pallas as pl
from jax.experimental.pallas import tpu as pltpu
from jax.experimental.pallas import tpu_sc as plsc

_ALPHA = 0.1
_BETA = math.log(0.5 / 1 + 1.0)

_N_NODES = 10000
_HIDDEN = 128

_NC = 2
_NS = 16
_NW = _NC * _NS
_CHUNK = 16
_CHUNKS_PER_W = 625
_EDGES_PER_W = _CHUNK * _CHUNKS_PER_W
_NBUF = 5
_ROWS_PER_TILE = 640
_ZCOPY = 16


def _sc_aggregate(xb, ei, nrm):
    mesh = plsc.VectorSubcoreMesh(core_axis_name="c", subcore_axis_name="s")

    def body(x_hbm, ei_hbm, nrm_hbm, out_hbm, rows5, stag, iring, nring,
             acc, xs, g0, g1, g2, g3, g4, e0, e1, e2, e3, e4):
        c = lax.axis_index("c")
        s = lax.axis_index("s")
        wid = s * _NC + c
        gsems = (g0, g1, g2, g3, g4)
        esems = (e0, e1, e2, e3, e4)

        def ecopy(k, slot):
            pltpu.async_copy(ei_hbm.at[0, wid, k], iring.at[slot, 0],
                             esems[slot])
            pltpu.async_copy(ei_hbm.at[1, wid, k], iring.at[slot, 1],
                             esems[slot])
            pltpu.async_copy(nrm_hbm.at[wid, k], nring.at[slot],
                             esems[slot])

        def ewait(k, slot):
            pltpu.make_async_copy(ei_hbm.at[0, wid, k], iring.at[slot, 0],
                                  esems[slot]).wait()
            pltpu.make_async_copy(ei_hbm.at[1, wid, k], iring.at[slot, 1],
                                  esems[slot]).wait()
            pltpu.make_async_copy(nrm_hbm.at[wid, k], nring.at[slot],
                                  esems[slot]).wait()

        def gather(slot):
            pltpu.async_copy(xs.at[iring.at[slot, 0]], rows5.at[slot],
                             gsems[slot])

        def gwait(slot):
            pltpu.make_async_copy(xs.at[iring.at[slot, 0]],
                                  rows5.at[slot], gsems[slot]).wait()

        def scatter(slot):
            pltpu.sync_copy(stag, acc.at[iring.at[slot, 1]], add=True)

        def scale(slot):
            def sg(g, c2):
                n16 = nring[slot, pl.ds(g * 16, 16)]
                for jj in range(16):
                    e = g * 16 + jj
                    nb = jnp.full((16,), n16[jj], jnp.float32)
                    for f in range(_HIDDEN // 32):
                        w = rows5[slot, e, pl.ds(f * 16, 16)]
                        lo = lax.bitcast_convert_type(w << 16, jnp.float32)
                        hi = lax.bitcast_convert_type(
                            w & jnp.full((16,), -65536, jnp.int32),
                            jnp.float32)
                        stag[e, pl.ds(f * 32, 16)] = lo * nb
                        stag[e, pl.ds(f * 32 + 16, 16)] = hi * nb
                return c2
            lax.fori_loop(0, _CHUNK // 16, sg, 0)

        def zrow(e, carry):
            for f in range(_HIDDEN // 16):
                stag[e, pl.ds(f * 16, 16)] = jnp.zeros((16,), jnp.float32)
            return carry
        lax.fori_loop(0, _CHUNK, zrow, 0)
        base = s * _ROWS_PER_TILE

        @pl.when(s < _NS - 1)
        def _():
            for i in range(_ROWS_PER_TILE // _ZCOPY):
                pltpu.sync_copy(stag, acc.at[pl.ds(base + i * _ZCOPY,
                                                   _ZCOPY)])

        @pl.when(s == _NS - 1)
        def _():
            for i in range(25):
                pltpu.sync_copy(
                    stag, acc.at[pl.ds(9600 + i * _ZCOPY, _ZCOPY)])

        pltpu.sync_copy(x_hbm.at[pl.ds(s * 625, 625)],
                        xs.at[pl.ds(s * 625, 625)])
        plsc.subcore_barrier()

        for kk in range(4):
            ecopy(kk, kk)
        for kk in range(3):
            ewait(kk, kk)
            gather(kk)

        _M = _CHUNKS_PER_W // _NBUF
        def ring(m, carry):
            for j in range(_NBUF):
                gwait(j)
                gslot = (j + 3) % _NBUF
                if j < 2:
                    ewait(5 * m + j + 3, gslot)
                    gather(gslot)
                else:
                    @pl.when(m < _M - 1)
                    def _():
                        ewait(5 * m + j + 3, gslot)
                        gather(gslot)
                if j < 1:
                    ecopy(5 * m + j + 4, (j + 4) % _NBUF)
                else:
                    @pl.when(m < _M - 1)
                    def _():
                        ecopy(5 * m + j + 4, (j + 4) % _NBUF)
                scale(j)
                scatter(j)
            return carry
        lax.fori_loop(0, _M, ring, 0)

        plsc.subcore_barrier()

        @pl.when(s < _NS - 1)
        def _():
            pltpu.sync_copy(acc.at[pl.ds(base, _ROWS_PER_TILE)],
                            out_hbm.at[c, pl.ds(base, _ROWS_PER_TILE)])

        @pl.when(s == _NS - 1)
        def _():
            pltpu.sync_copy(acc.at[pl.ds(9600, 400)],
                            out_hbm.at[c, pl.ds(9600, 400)])

    return pl.kernel(
        body,
        out_type=jax.ShapeDtypeStruct((_NC, _N_NODES, _HIDDEN), jnp.float32),
        mesh=mesh,
        compiler_params=pltpu.CompilerParams(use_tc_tiling_on_sc=False),
        scratch_types=[
            pltpu.VMEM((_NBUF, _CHUNK, _HIDDEN // 2), jnp.int32),
            pltpu.VMEM((_CHUNK, _HIDDEN), jnp.float32),
            pltpu.VMEM((_NBUF, 2, _CHUNK), jnp.int32),
            pltpu.VMEM((_NBUF, _CHUNK), jnp.float32),
            pltpu.VMEM_SHARED((_N_NODES, _HIDDEN), jnp.float32),
            pltpu.VMEM_SHARED((_N_NODES, _HIDDEN // 2), jnp.int32),
        ] + [pltpu.SemaphoreType.DMA] * 10,
    )(xb, ei, nrm)


def _combine_body(p_ref, x0_ref, w_ref, o_ref):
    p = p_ref[...]
    h = (1.0 - _ALPHA) * (p[0] + p[1]) + _ALPHA * x0_ref[...]
    hw = lax.dot_general(h, w_ref[...], (((1,), (1,)), ((), ())),
                         preferred_element_type=jnp.float32)
    o_ref[...] = (1.0 - _BETA) * h + _BETA * hw


def _tc_combine(partials, x0, W):
    blk = 400
    grid = _N_NODES // blk
    return pl.pallas_call(
        _combine_body,
        grid=(grid,),
        in_specs=[
            pl.BlockSpec((_NC, blk, _HIDDEN), lambda i: (0, i, 0)),
            pl.BlockSpec((blk, _HIDDEN), lambda i: (i, 0)),
            pl.BlockSpec((_HIDDEN, _HIDDEN), lambda i: (0, 0)),
        ],
        out_specs=pl.BlockSpec((blk, _HIDDEN), lambda i: (i, 0)),
        out_shape=jax.ShapeDtypeStruct((_N_NODES, _HIDDEN), jnp.float32),
    )(partials, x0, W)


def kernel(x, x0, edge_index, norm, W):
    ei = edge_index.astype(jnp.int32).reshape(
        2, _NW, _CHUNKS_PER_W, _CHUNK)
    nrm = norm.astype(jnp.float32).reshape(_NW, _CHUNKS_PER_W, _CHUNK)
    xb = lax.bitcast_convert_type(
        x.astype(jnp.bfloat16).reshape(
            _N_NODES, _HIDDEN // 32, 2, 16).transpose(0, 1, 3, 2),
        jnp.int32).reshape(_N_NODES, _HIDDEN // 2)
    partials = _sc_aggregate(xb, ei, nrm)
    return _tc_combine(partials, x0, W)
